# baseline (speedup 1.0000x reference)
import functools

import jax
import jax.numpy as jnp
from jax import lax
from jax.experimental import pallas as pl
from jax.experimental.pallas import tpu as pltpu
from jax.experimental.pallas import tpu_sc as plsc

_RAD2 = 0.36
_NSAMPLE = 32
_LANES = 16
_C_OUT = 128
_GRP = 8


def _mlp_body(x_ref, w1_ref, b1_ref, w2_ref, b2_ref, out_ref):
    x = x_ref[...]
    h1 = jnp.dot(x, w1_ref[...], preferred_element_type=jnp.float32,
                 precision=lax.Precision.HIGHEST)
    h1 = jnp.maximum(h1 + b1_ref[...], 0.0)
    h2 = jnp.dot(h1, w2_ref[...], preferred_element_type=jnp.float32,
                 precision=lax.Precision.HIGHEST)
    out_ref[...] = jnp.maximum(h2 + b2_ref[...], 0.0)


def _mlp(xp, w1p, b1, w2t, b2):
    bn = xp.shape[0]
    tile = 1024
    grid = bn // tile
    return pl.pallas_call(
        _mlp_body,
        grid=(grid,),
        in_specs=[
            pl.BlockSpec((tile, xp.shape[1]), lambda i: (i, 0)),
            pl.BlockSpec(w1p.shape, lambda i: (0, 0)),
            pl.BlockSpec(b1.shape, lambda i: (0, 0)),
            pl.BlockSpec(w2t.shape, lambda i: (0, 0)),
            pl.BlockSpec(b2.shape, lambda i: (0, 0)),
        ],
        out_specs=pl.BlockSpec((tile, _C_OUT), lambda i: (i, 0)),
        out_shape=jax.ShapeDtypeStruct((bn, _C_OUT), jnp.float32),
    )(xp, w1p, b1, w2t, b2)


def _sc_scan(xs, ys, zs, batch, n):
    info = plsc.get_sparse_core_info()
    nw = info.num_cores * info.num_subcores
    bn = batch * n
    nq = bn // nw
    mesh = plsc.VectorSubcoreMesh(core_axis_name="c", subcore_axis_name="s")

    @functools.partial(
        pl.kernel,
        mesh=mesh,
        compiler_params=pltpu.CompilerParams(needs_layout_passes=False),
        out_type=jax.ShapeDtypeStruct((bn * _NSAMPLE,), jnp.int32),
        scratch_types=[
            pltpu.VMEM((n,), jnp.float32),
            pltpu.VMEM((n,), jnp.float32),
            pltpu.VMEM((n,), jnp.float32),
            pltpu.VMEM((128,), jnp.int32),
            pltpu.VMEM((nq * _NSAMPLE,), jnp.int32),
        ],
    )
    def k(xs_h, ys_h, zs_h, idx_h, xs_v, ys_v, zs_v, found, idxbuf):
        cid = lax.axis_index("c")
        sid = lax.axis_index("s")
        wid = cid * info.num_subcores + sid
        qg0 = wid * nq
        b = qg0 // n
        base = b * n
        core_base = cid * (bn // info.num_cores)
        pltpu.sync_copy(xs_h.at[pl.ds(base, n)], xs_v)
        pltpu.sync_copy(ys_h.at[pl.ds(base, n)], ys_v)
        pltpu.sync_copy(zs_h.at[pl.ds(base, n)], zs_v)

        def per_query(qi, carry):
            q = (qg0 - base) + qi
            ql = qg0 - core_base + qi
            qv = jnp.full((_LANES,), q, jnp.int32)
            xq = plsc.load_gather(xs_v, [qv])
            yq = plsc.load_gather(ys_v, [qv])
            zq = plsc.load_gather(zs_v, [qv])
            pad = jnp.full((_LANES,), ql, jnp.int32)
            found[pl.ds(0, _LANES)] = pad
            found[pl.ds(_LANES, _LANES)] = pad

            def cond(st):
                jb, cnt = st
                return jnp.logical_and(cnt < _NSAMPLE, jb < n)

            def chunk(jb, cnt):
                jv = jb + lax.iota(jnp.int32, _LANES)
                dx = xs_v[pl.ds(jb, _LANES)] - xq
                dy = ys_v[pl.ds(jb, _LANES)] - yq
                dz = zs_v[pl.ds(jb, _LANES)] - zq
                sq = dx * dx + dy * dy + dz * dz
                m = sq <= _RAD2
                plsc.store_compressed(found.at[pl.ds(cnt, _LANES)],
                                      jv + (base - core_base), mask=m)
                return cnt + plsc.all_reduce_population_count(m)[0]

            def body(st):
                jb, cnt = st
                cnt = chunk(jb, cnt)
                cnt = chunk(jb + _LANES, cnt)
                return jb + 2 * _LANES, cnt

            cnt0 = jnp.int32(0)
            for j in range(6):
                cnt0 = chunk(jnp.int32(j * _LANES), cnt0)
            lax.while_loop(cond, body, (jnp.int32(6 * _LANES), cnt0))

            off = qi * _NSAMPLE
            idxbuf[pl.ds(off, _LANES)] = found[pl.ds(0, _LANES)]
            idxbuf[pl.ds(off + _LANES, _LANES)] = found[pl.ds(_LANES, _LANES)]
            return carry

        lax.fori_loop(0, nq, per_query, jnp.int32(0))
        pltpu.sync_copy(idxbuf, idx_h.at[pl.ds(qg0 * _NSAMPLE, nq * _NSAMPLE)])

    return k(xs, ys, zs)


def _sc_gathermax(h2, idx, batch, n):
    info = plsc.get_sparse_core_info()
    nw = info.num_cores * info.num_subcores
    bn = batch * n
    nq = bn // nw
    ngroups = nq // _GRP
    nhalf = nq // 2
    mesh = plsc.VectorSubcoreMesh(core_axis_name="c", subcore_axis_name="s")

    @functools.partial(
        pl.kernel,
        mesh=mesh,
        compiler_params=pltpu.CompilerParams(needs_layout_passes=False),
        out_type=jax.ShapeDtypeStruct((bn, _C_OUT), jnp.float32),
        scratch_types=[
            pltpu.VMEM((nq * _NSAMPLE,), jnp.int32),
            pltpu.VMEM((_GRP * _NSAMPLE, _C_OUT), jnp.float32),
            pltpu.VMEM((_GRP * _NSAMPLE, _C_OUT), jnp.float32),
            pltpu.VMEM((nhalf, _C_OUT), jnp.float32),
            pltpu.VMEM_SHARED((bn // 2, _C_OUT), jnp.float32),
            pltpu.SemaphoreType.DMA,
            pltpu.SemaphoreType.DMA,
        ],
    )
    def k(h2_h, idx_h, out_h, idx_v, rows_a, rows_b, outbuf, h2_s,
          sem_a, sem_b):
        cid = lax.axis_index("c")
        sid = lax.axis_index("s")
        wid = cid * info.num_subcores + sid
        qg0 = wid * nq
        core_rows = bn // info.num_cores
        core_base = cid * core_rows
        slab = core_rows // info.num_subcores
        pltpu.sync_copy(h2_h.at[pl.ds(core_base + sid * slab, slab)],
                        h2_s.at[pl.ds(sid * slab, slab)])
        pltpu.sync_copy(idx_h.at[pl.ds(qg0 * _NSAMPLE, nq * _NSAMPLE)], idx_v)
        plsc.subcore_barrier()

        def start(g, rows_ref, sem):
            pltpu.make_async_copy(
                h2_s.at[idx_v.at[pl.ds(g * _GRP * _NSAMPLE, _GRP * _NSAMPLE)]],
                rows_ref, sem).start()

        def wait(g, rows_ref, sem):
            pltpu.make_async_copy(
                h2_s.at[idx_v.at[pl.ds(g * _GRP * _NSAMPLE, _GRP * _NSAMPLE)]],
                rows_ref, sem).wait()

        nchunk = _C_OUT // _LANES

        def reduce_group(g, rows_ref):
            for t in range(_GRP):
                r0 = t * _NSAMPLE

                def rmax(r, a):
                    return tuple(
                        jnp.maximum(a[c],
                                    rows_ref[r, pl.ds(c * _LANES, _LANES)])
                        for c in range(nchunk))

                accs = tuple(rows_ref[r0, pl.ds(c * _LANES, _LANES)]
                             for c in range(nchunk))

                def body(k_, a):
                    r = r0 + 1 + 4 * k_
                    for d in range(4):
                        a = rmax(r + d, a)
                    return a

                accs = lax.fori_loop(0, 7, body, accs)
                for d in range(29, _NSAMPLE):
                    accs = rmax(r0 + d, accs)
                row = jnp.bitwise_and(g * _GRP + t, nhalf - 1)
                for c in range(nchunk):
                    outbuf[row, pl.ds(c * _LANES, _LANES)] = accs[c]

        start(0, rows_a, sem_a)
        flush_k = nhalf // _GRP // 2 - 1

        def pair(k_, carry):
            g = 2 * k_
            start(g + 1, rows_b, sem_b)
            wait(g, rows_a, sem_a)
            reduce_group(g, rows_a)
            start(g + 2, rows_a, sem_a)
            wait(g + 1, rows_b, sem_b)
            reduce_group(g + 1, rows_b)

            @pl.when(k_ == flush_k)
            def _():
                pltpu.sync_copy(outbuf, out_h.at[pl.ds(qg0, nhalf)])

            return carry

        lax.fori_loop(0, ngroups // 2 - 1, pair, jnp.int32(0))

        gl = ngroups - 2
        start(gl + 1, rows_b, sem_b)
        wait(gl, rows_a, sem_a)
        reduce_group(gl, rows_a)
        wait(gl + 1, rows_b, sem_b)
        reduce_group(gl + 1, rows_b)

        pltpu.sync_copy(outbuf, out_h.at[pl.ds(qg0 + nhalf, nhalf)])

    return k(h2, idx)


def kernel(x, W1, b1, W2, b2):
    batch, n, _ = x.shape
    bn = batch * n
    xf = x.reshape(bn, 3)
    xp = jnp.pad(xf, ((0, 0), (0, 5)))
    w1p = jnp.pad(W1.T, ((0, 5), (0, 0)))
    idx = _sc_scan(xf[:, 0], xf[:, 1], xf[:, 2], batch, n)
    h2 = _mlp(xp, w1p, b1.reshape(1, -1), W2.T, b2.reshape(1, -1))
    out_t = _sc_gathermax(h2, idx, batch, n)
    return out_t.reshape(batch, n, _C_OUT).transpose(0, 2, 1)


# speedup vs baseline: 1.0585x; 1.0585x over previous
import functools

import jax
import jax.numpy as jnp
from jax import lax
from jax.experimental import pallas as pl
from jax.experimental.pallas import tpu as pltpu
from jax.experimental.pallas import tpu_sc as plsc

_RAD2 = 0.36
_NSAMPLE = 32
_LANES = 16
_C_OUT = 128
_GRP = 4


def _mlp_body(x_ref, w1_ref, b1_ref, w2_ref, b2_ref, out_ref):
    x = x_ref[...]
    h1 = jnp.dot(x, w1_ref[...], preferred_element_type=jnp.float32,
                 precision=lax.Precision.HIGHEST)
    h1 = jnp.maximum(h1 + b1_ref[...], 0.0)
    h2 = jnp.dot(h1, w2_ref[...], preferred_element_type=jnp.float32,
                 precision=lax.Precision.HIGHEST)
    out_ref[...] = jnp.maximum(h2 + b2_ref[...], 0.0)


def _mlp(xp, w1p, b1, w2t, b2):
    bn = xp.shape[0]
    tile = 1024
    grid = bn // tile
    return pl.pallas_call(
        _mlp_body,
        grid=(grid,),
        in_specs=[
            pl.BlockSpec((tile, xp.shape[1]), lambda i: (i, 0)),
            pl.BlockSpec(w1p.shape, lambda i: (0, 0)),
            pl.BlockSpec(b1.shape, lambda i: (0, 0)),
            pl.BlockSpec(w2t.shape, lambda i: (0, 0)),
            pl.BlockSpec(b2.shape, lambda i: (0, 0)),
        ],
        out_specs=pl.BlockSpec((tile, _C_OUT), lambda i: (i, 0)),
        out_shape=jax.ShapeDtypeStruct((bn, _C_OUT), jnp.float32),
    )(xp, w1p, b1, w2t, b2)


def _sc_scan(xs, ys, zs, batch, n):
    info = plsc.get_sparse_core_info()
    nw = info.num_cores * info.num_subcores
    bn = batch * n
    nq = bn // nw
    mesh = plsc.VectorSubcoreMesh(core_axis_name="c", subcore_axis_name="s")

    @functools.partial(
        pl.kernel,
        mesh=mesh,
        compiler_params=pltpu.CompilerParams(needs_layout_passes=False),
        out_type=jax.ShapeDtypeStruct((bn * _NSAMPLE,), jnp.int32),
        scratch_types=[
            pltpu.VMEM((n,), jnp.float32),
            pltpu.VMEM((n,), jnp.float32),
            pltpu.VMEM((n,), jnp.float32),
            pltpu.VMEM((nq * _NSAMPLE + 128,), jnp.int32),
        ],
    )
    def k(xs_h, ys_h, zs_h, idx_h, xs_v, ys_v, zs_v, idxbuf):
        cid = lax.axis_index("c")
        sid = lax.axis_index("s")
        wid = cid * info.num_subcores + sid
        qg0 = wid * nq
        b = qg0 // n
        base = b * n
        core_base = cid * (bn // info.num_cores)
        pltpu.sync_copy(xs_h.at[pl.ds(base, n)], xs_v)
        pltpu.sync_copy(ys_h.at[pl.ds(base, n)], ys_v)
        pltpu.sync_copy(zs_h.at[pl.ds(base, n)], zs_v)

        def per_query(qi, carry):
            q = (qg0 - base) + qi
            ql = qg0 - core_base + qi
            qv = jnp.full((_LANES,), q, jnp.int32)
            xq = plsc.load_gather(xs_v, [qv])
            yq = plsc.load_gather(ys_v, [qv])
            zq = plsc.load_gather(zs_v, [qv])
            off = qi * _NSAMPLE
            pad = jnp.full((_LANES,), ql, jnp.int32)
            idxbuf[pl.ds(off, _LANES)] = pad
            idxbuf[pl.ds(off + _LANES, _LANES)] = pad

            def cond(st):
                jb, cnt = st
                return jnp.logical_and(cnt < _NSAMPLE, jb < n)

            def chunk(jb, cnt):
                jv = jb + lax.iota(jnp.int32, _LANES)
                dx = xs_v[pl.ds(jb, _LANES)] - xq
                dy = ys_v[pl.ds(jb, _LANES)] - yq
                dz = zs_v[pl.ds(jb, _LANES)] - zq
                sq = dx * dx + dy * dy + dz * dz
                m = sq <= _RAD2
                plsc.store_compressed(idxbuf.at[pl.ds(off + cnt, _LANES)],
                                      jv + (base - core_base), mask=m)
                return cnt + plsc.all_reduce_population_count(m)[0]

            def body(st):
                jb, cnt = st
                cnt = chunk(jb, cnt)
                cnt = chunk(jb + _LANES, cnt)
                return jb + 2 * _LANES, cnt

            cnt0 = jnp.int32(0)
            for j in range(6):
                cnt0 = chunk(jnp.int32(j * _LANES), cnt0)
            lax.while_loop(cond, body, (jnp.int32(6 * _LANES), cnt0))
            return carry

        lax.fori_loop(0, nq, per_query, jnp.int32(0))
        pltpu.sync_copy(idxbuf.at[pl.ds(0, nq * _NSAMPLE)],
                        idx_h.at[pl.ds(qg0 * _NSAMPLE, nq * _NSAMPLE)])

    return k(xs, ys, zs)


def _sc_gathermax(h2, idx, batch, n):
    info = plsc.get_sparse_core_info()
    nw = info.num_cores * info.num_subcores
    bn = batch * n
    nq = bn // nw
    ngroups = nq // _GRP
    mesh = plsc.VectorSubcoreMesh(core_axis_name="c", subcore_axis_name="s")

    @functools.partial(
        pl.kernel,
        mesh=mesh,
        compiler_params=pltpu.CompilerParams(needs_layout_passes=False),
        out_type=jax.ShapeDtypeStruct((bn, _C_OUT), jnp.float32),
        scratch_types=[
            pltpu.VMEM((nq * _NSAMPLE,), jnp.int32),
            pltpu.VMEM((_GRP * _NSAMPLE, _C_OUT), jnp.float32),
            pltpu.VMEM((_GRP * _NSAMPLE, _C_OUT), jnp.float32),
            pltpu.VMEM((nq, _C_OUT), jnp.float32),
            pltpu.VMEM_SHARED((bn // 2, _C_OUT), jnp.float32),
            pltpu.SemaphoreType.DMA,
            pltpu.SemaphoreType.DMA,
        ],
    )
    def k(h2_h, idx_h, out_h, idx_v, rows_a, rows_b, outbuf, h2_s,
          sem_a, sem_b):
        cid = lax.axis_index("c")
        sid = lax.axis_index("s")
        wid = cid * info.num_subcores + sid
        qg0 = wid * nq
        core_rows = bn // info.num_cores
        core_base = cid * core_rows
        slab = core_rows // info.num_subcores
        pltpu.sync_copy(h2_h.at[pl.ds(core_base + sid * slab, slab)],
                        h2_s.at[pl.ds(sid * slab, slab)])
        pltpu.sync_copy(idx_h.at[pl.ds(qg0 * _NSAMPLE, nq * _NSAMPLE)], idx_v)
        plsc.subcore_barrier()

        def start(g, rows_ref, sem):
            pltpu.make_async_copy(
                h2_s.at[idx_v.at[pl.ds(g * _GRP * _NSAMPLE, _GRP * _NSAMPLE)]],
                rows_ref, sem).start()

        def wait(g, rows_ref, sem):
            pltpu.make_async_copy(
                h2_s.at[idx_v.at[pl.ds(g * _GRP * _NSAMPLE, _GRP * _NSAMPLE)]],
                rows_ref, sem).wait()

        nchunk = _C_OUT // _LANES

        def reduce_group(g, rows_ref):
            for t in range(_GRP):
                r0 = t * _NSAMPLE

                def rmax(r, a):
                    return tuple(
                        jnp.maximum(a[c],
                                    rows_ref[r, pl.ds(c * _LANES, _LANES)])
                        for c in range(nchunk))

                accs = tuple(rows_ref[r0, pl.ds(c * _LANES, _LANES)]
                             for c in range(nchunk))

                def body(k_, a):
                    r = r0 + 1 + 4 * k_
                    for d in range(4):
                        a = rmax(r + d, a)
                    return a

                accs = lax.fori_loop(0, 7, body, accs)
                for d in range(29, _NSAMPLE):
                    accs = rmax(r0 + d, accs)
                for c in range(nchunk):
                    outbuf[g * _GRP + t, pl.ds(c * _LANES, _LANES)] = accs[c]

        start(0, rows_a, sem_a)

        def pair(k_, carry):
            g = 2 * k_
            start(g + 1, rows_b, sem_b)
            wait(g, rows_a, sem_a)
            reduce_group(g, rows_a)
            start(g + 2, rows_a, sem_a)
            wait(g + 1, rows_b, sem_b)
            reduce_group(g + 1, rows_b)
            return carry

        lax.fori_loop(0, ngroups // 2 - 1, pair, jnp.int32(0))

        gl = ngroups - 2
        start(gl + 1, rows_b, sem_b)
        wait(gl, rows_a, sem_a)
        reduce_group(gl, rows_a)
        wait(gl + 1, rows_b, sem_b)
        reduce_group(gl + 1, rows_b)

        pltpu.sync_copy(outbuf, out_h.at[pl.ds(qg0, nq)])

    return k(h2, idx)


def kernel(x, W1, b1, W2, b2):
    batch, n, _ = x.shape
    bn = batch * n
    xf = x.reshape(bn, 3)
    xp = jnp.pad(xf, ((0, 0), (0, 5)))
    w1p = jnp.pad(W1.T, ((0, 5), (0, 0)))
    idx = _sc_scan(xf[:, 0], xf[:, 1], xf[:, 2], batch, n)
    h2 = _mlp(xp, w1p, b1.reshape(1, -1), W2.T, b2.reshape(1, -1))
    out_t = _sc_gathermax(h2, idx, batch, n)
    return out_t.reshape(batch, n, _C_OUT).transpose(0, 2, 1)
